# baseline (speedup 1.0000x reference)
import functools

import jax
import jax.numpy as jnp
from jax import lax
from jax.experimental import pallas as pl
from jax.experimental.pallas import tpu as pltpu
from jax.experimental.pallas import tpu_sc as plsc

_NUM_CORES = 2
_NUM_SUBCORES = 16
_NW = _NUM_CORES * _NUM_SUBCORES

_NBUF = 3


def _round_up(x: int, m: int) -> int:
    return (x + m - 1) // m * m


@functools.lru_cache(maxsize=None)
def _make_gather(num_rows: int, dim: int, bsz: int, fields: int):
    assert bsz % _NW == 0
    fpad = _round_up(fields, 8)
    ppw = bsz // _NW
    ipw = ppw * fpad
    ppc = 4
    assert ppw % ppc == 0
    nch = ppw // ppc
    rows = ppc * fpad

    mesh = plsc.VectorSubcoreMesh(core_axis_name="c", subcore_axis_name="s")

    @functools.partial(
        pl.kernel,
        out_type=jax.ShapeDtypeStruct((bsz, fpad, dim), jnp.float32),
        mesh=mesh,
        compiler_params=pltpu.CompilerParams(use_tc_tiling_on_sc=True),
        scratch_types=[
            pltpu.VMEM((ipw,), jnp.int32),
            [pltpu.VMEM((rows, dim), jnp.float32) for _ in range(_NBUF)],
            [pltpu.SemaphoreType.DMA for _ in range(_NBUF)],
            [pltpu.SemaphoreType.DMA for _ in range(_NBUF)],
        ],
    )
    def gather(idx_hbm, tbl_hbm, out_hbm, idx_v, bufs, gsems, wsems):
        wid = lax.axis_index("s") * _NUM_CORES + lax.axis_index("c")
        pbase = wid * ppw
        pltpu.sync_copy(idx_hbm.at[pl.ds(wid * ipw, ipw)], idx_v)

        def write_out(g):
            return pltpu.async_copy(
                bufs[g % _NBUF].reshape(ppc, fpad, dim),
                out_hbm.at[pl.ds(pbase + g * ppc, ppc)],
                wsems[g % _NBUF])

        gd = [None] * nch
        wd = [None] * nch
        for g in range(nch):
            b = g % _NBUF
            if g >= _NBUF:
                wd[g - _NBUF].wait()
            gd[g] = pltpu.async_copy(
                tbl_hbm.at[idx_v.at[pl.ds(g * rows, rows)]],
                bufs[b], gsems[b])
            if g >= 1:
                gd[g - 1].wait()
                wd[g - 1] = write_out(g - 1)
        gd[nch - 1].wait()
        wd[nch - 1] = write_out(nch - 1)
        for g in range(max(0, nch - _NBUF + 1), nch):
            wd[g].wait()

    return gather


def _mm_body(a_ref, b_ref, o_ref):
    o_ref[...] = jnp.dot(a_ref[...], b_ref[...],
                         preferred_element_type=jnp.float32)


@functools.lru_cache(maxsize=None)
def _make_mm(m: int, k: int, n: int):
    return pl.pallas_call(
        _mm_body,
        out_shape=jax.ShapeDtypeStruct((m, n), jnp.float32),
    )


@jax.jit
def kernel(input_batch_inputs_, weight, mat1, mat2):
    bsz, fields = input_batch_inputs_.shape
    num_rows, dim = weight.shape
    fpad = _round_up(fields, 8)
    idx_pad = jnp.pad(input_batch_inputs_,
                      ((0, 0), (0, fpad - fields))).reshape(-1)
    emb_pad = _make_gather(num_rows, dim, bsz, fields)(idx_pad, weight)
    emb = emb_pad[:, :fields, :]
    mm = _make_mm(mat1.shape[0], mat1.shape[1], mat2.shape[1])(mat1, mat2)
    return emb, mm


# speedup vs baseline: 1.0015x; 1.0015x over previous
import functools

import jax
import jax.numpy as jnp
from jax import lax
from jax.experimental import pallas as pl
from jax.experimental.pallas import tpu as pltpu
from jax.experimental.pallas import tpu_sc as plsc

_NUM_CORES = 2
_NUM_SUBCORES = 16
_NW = _NUM_CORES * _NUM_SUBCORES

_NBUF = 3


def _round_up(x: int, m: int) -> int:
    return (x + m - 1) // m * m


@functools.lru_cache(maxsize=None)
def _make_gather(num_rows: int, dim: int, bsz: int, fields: int):
    assert bsz % _NW == 0
    fpad = _round_up(fields, 8)
    ppw = bsz // _NW
    ipw = ppw * fpad
    ppc = 4
    assert ppw % ppc == 0
    nch = ppw // ppc
    rows = ppc * fpad

    mesh = plsc.VectorSubcoreMesh(core_axis_name="c", subcore_axis_name="s")

    @functools.partial(
        pl.kernel,
        out_type=jax.ShapeDtypeStruct((bsz, fpad, dim), jnp.float32),
        mesh=mesh,
        compiler_params=pltpu.CompilerParams(use_tc_tiling_on_sc=True),
        scratch_types=[
            pltpu.VMEM((ipw,), jnp.int32),
            [pltpu.VMEM((rows, dim), jnp.float32) for _ in range(_NBUF)],
            [pltpu.SemaphoreType.DMA for _ in range(_NBUF)],
            [pltpu.SemaphoreType.DMA for _ in range(_NBUF)],
        ],
    )
    def gather(idx_hbm, tbl_hbm, out_hbm, idx_v, bufs, gsems, wsems):
        wid = lax.axis_index("s") * _NUM_CORES + lax.axis_index("c")
        pbase = wid * ppw
        pltpu.sync_copy(idx_hbm.at[pl.ds(wid * ipw, ipw)], idx_v)

        def write_out(g):
            ds = []
            for q in range(ppc):
                ds.append(pltpu.async_copy(
                    bufs[g % _NBUF].at[pl.ds(q * fpad, fpad)],
                    out_hbm.at[pbase + g * ppc + q],
                    wsems[g % _NBUF]))
            return ds

        gd = [None] * nch
        wd = [None] * nch
        for g in range(nch):
            b = g % _NBUF
            if g >= _NBUF:
                for d in wd[g - _NBUF]:
                    d.wait()
            gd[g] = pltpu.async_copy(
                tbl_hbm.at[idx_v.at[pl.ds(g * rows, rows)]],
                bufs[b], gsems[b])
            if g >= 1:
                gd[g - 1].wait()
                wd[g - 1] = write_out(g - 1)
        gd[nch - 1].wait()
        wd[nch - 1] = write_out(nch - 1)
        for g in range(max(0, nch - _NBUF + 1), nch):
            for d in wd[g]:
                d.wait()

    return gather


def _mm_body(a_ref, b_ref, o_ref):
    o_ref[...] = jnp.dot(a_ref[...], b_ref[...],
                         preferred_element_type=jnp.float32)


@functools.lru_cache(maxsize=None)
def _make_mm(m: int, k: int, n: int):
    return pl.pallas_call(
        _mm_body,
        out_shape=jax.ShapeDtypeStruct((m, n), jnp.float32),
    )


@jax.jit
def kernel(input_batch_inputs_, weight, mat1, mat2):
    bsz, fields = input_batch_inputs_.shape
    num_rows, dim = weight.shape
    fpad = _round_up(fields, 8)
    idx_pad = jnp.pad(input_batch_inputs_,
                      ((0, 0), (0, fpad - fields))).reshape(-1)
    emb_pad = _make_gather(num_rows, dim, bsz, fields)(idx_pad, weight)
    emb = emb_pad[:, :fields, :]
    mm = _make_mm(mat1.shape[0], mat1.shape[1], mat2.shape[1])(mat1, mat2)
    return emb, mm


# speedup vs baseline: 6.5960x; 6.5858x over previous
import functools

import jax
import jax.numpy as jnp
from jax import lax
from jax.experimental import pallas as pl
from jax.experimental.pallas import tpu as pltpu
from jax.experimental.pallas import tpu_sc as plsc

_NUM_CORES = 2
_NUM_SUBCORES = 16
_NW = _NUM_CORES * _NUM_SUBCORES

_NBUF = 3


def _round_up(x: int, m: int) -> int:
    return (x + m - 1) // m * m


@functools.lru_cache(maxsize=None)
def _make_gather(num_rows: int, dim: int, bsz: int, fields: int):
    assert bsz % _NW == 0
    fpad = _round_up(fields, 8)
    ppw = bsz // _NW
    ipw = ppw * fpad
    ppc = 4
    assert ppw % ppc == 0
    nch = ppw // ppc
    rows = ppc * fpad

    mesh = plsc.VectorSubcoreMesh(core_axis_name="c", subcore_axis_name="s")

    @functools.partial(
        pl.kernel,
        out_type=jax.ShapeDtypeStruct((bsz, fpad, dim), jnp.float32),
        mesh=mesh,
        compiler_params=pltpu.CompilerParams(use_tc_tiling_on_sc=True),
        scratch_types=[
            pltpu.VMEM((ipw,), jnp.int32),
            [pltpu.VMEM((rows, dim), jnp.float32) for _ in range(_NBUF)],
            [pltpu.SemaphoreType.DMA for _ in range(_NBUF)],
            [pltpu.SemaphoreType.DMA for _ in range(_NBUF)],
        ],
    )
    def gather(idx_hbm, tbl_hbm, out_hbm, idx_v, bufs, gsems, wsems):
        wid = lax.axis_index("s") * _NUM_CORES + lax.axis_index("c")
        pbase = wid * ppw
        pltpu.sync_copy(idx_hbm.at[pl.ds(wid * ipw, ipw)], idx_v)

        def write_out(g):
            ds = []
            for q in range(ppc):
                ds.append(pltpu.async_copy(
                    bufs[g % _NBUF].at[pl.ds(q * fpad, fpad)],
                    out_hbm.at[pbase + g * ppc + q],
                    wsems[g % _NBUF]))
            return ds

        gd = [None] * nch
        wd = [None] * nch
        for g in range(nch):
            b = g % _NBUF
            if g >= _NBUF:
                for d in wd[g - _NBUF]:
                    d.wait()
            gd[g] = pltpu.async_copy(
                tbl_hbm.at[idx_v.at[pl.ds(g * rows, rows)]],
                bufs[b], gsems[b])
            if g >= 1:
                gd[g - 1].wait()
                wd[g - 1] = write_out(g - 1)
        gd[nch - 1].wait()
        wd[nch - 1] = write_out(nch - 1)
        for g in range(max(0, nch - _NBUF + 1), nch):
            for d in wd[g]:
                d.wait()

    return gather


def _mm_body(a_ref, b_ref, o_ref):
    o_ref[...] = jnp.dot(a_ref[...], b_ref[...],
                         preferred_element_type=jnp.float32)


@functools.lru_cache(maxsize=None)
def _make_mm(m: int, k: int, n: int):
    return pl.pallas_call(
        _mm_body,
        out_shape=jax.ShapeDtypeStruct((m, n), jnp.float32),
    )


@jax.jit
def kernel(input_batch_inputs_, weight, mat1, mat2):
    bsz, fields = input_batch_inputs_.shape
    num_rows, dim = weight.shape
    fpad = _round_up(fields, 8)
    idx_pad = jnp.pad(input_batch_inputs_,
                      ((0, 0), (0, fpad - fields)), mode="wrap").reshape(-1)
    emb_pad = _make_gather(num_rows, dim, bsz, fields)(idx_pad, weight)
    emb = emb_pad[:, :fields, :]
    mm = _make_mm(mat1.shape[0], mat1.shape[1], mat2.shape[1])(mat1, mat2)
    return emb, mm


# speedup vs baseline: 6.9787x; 1.0580x over previous
import functools

import jax
import jax.numpy as jnp
from jax import lax
from jax.experimental import pallas as pl
from jax.experimental.pallas import tpu as pltpu
from jax.experimental.pallas import tpu_sc as plsc

_NUM_CORES = 2
_NUM_SUBCORES = 16
_NW = _NUM_CORES * _NUM_SUBCORES

_NBUF = 8


def _round_up(x: int, m: int) -> int:
    return (x + m - 1) // m * m


@functools.lru_cache(maxsize=None)
def _make_gather(num_rows: int, dim: int, bsz: int, fields: int):
    assert bsz % _NW == 0
    assert dim % 128 == 0
    nh = dim // 128
    fpad = _round_up(fields, 8)
    ppw = bsz // _NW
    ipw = ppw * fpad

    mesh = plsc.VectorSubcoreMesh(core_axis_name="c", subcore_axis_name="s")

    @functools.partial(
        pl.kernel,
        out_type=jax.ShapeDtypeStruct((bsz, fields, dim), jnp.float32),
        mesh=mesh,
        compiler_params=pltpu.CompilerParams(use_tc_tiling_on_sc=True),
        scratch_types=[
            pltpu.VMEM((ipw,), jnp.int32),
            [[pltpu.VMEM((fields, 128), jnp.float32) for _ in range(nh)]
             for _ in range(_NBUF)],
            [pltpu.SemaphoreType.DMA for _ in range(_NBUF)],
            [pltpu.SemaphoreType.DMA for _ in range(_NBUF)],
        ],
    )
    def gather(idx_hbm, tbl_hbm, out_hbm, idx_v, bufs, gsems, wsems):
        wid = lax.axis_index("s") * _NUM_CORES + lax.axis_index("c")
        pbase = wid * ppw
        pltpu.sync_copy(idx_hbm.at[pl.ds(wid * ipw, ipw)], idx_v)

        def start_gather(p, b):
            ds = []
            for h in range(nh):
                ds.append(pltpu.async_copy(
                    tbl_hbm.at[idx_v.at[pl.ds(pl.multiple_of(p * fpad, 8),
                                              fields)],
                               pl.ds(h * 128, 128)],
                    bufs[b][h], gsems[b]))
            return ds

        def write_out(p, b):
            for h in range(nh):
                pltpu.async_copy(
                    bufs[b][h],
                    out_hbm.at[pbase + p, slice(None), pl.ds(h * 128, 128)],
                    wsems[b])

        def drain_writes(b):
            for h in range(nh):
                pltpu.make_async_copy(
                    bufs[b][h],
                    out_hbm.at[pbase, slice(None), pl.ds(h * 128, 128)],
                    wsems[b]).wait()

        rounds = ppw // _NBUF
        assert ppw % _NBUF == 0

        def body(r, carry):
            @pl.when(r > 0)
            def _():
                for b in range(_NBUF):
                    drain_writes(b)
            gds = [start_gather(r * _NBUF + b, b) for b in range(_NBUF)]
            for b in range(_NBUF):
                for d in gds[b]:
                    d.wait()
                write_out(r * _NBUF + b, b)
            return carry

        lax.fori_loop(0, rounds, body, 0, unroll=False)
        for b in range(_NBUF):
            drain_writes(b)

    return gather


def _mm_body(a_ref, b_ref, o_ref):
    o_ref[...] = jnp.dot(a_ref[...], b_ref[...],
                         preferred_element_type=jnp.float32)


@functools.lru_cache(maxsize=None)
def _make_mm(m: int, k: int, n: int):
    return pl.pallas_call(
        _mm_body,
        out_shape=jax.ShapeDtypeStruct((m, n), jnp.float32),
    )


@jax.jit
def kernel(input_batch_inputs_, weight, mat1, mat2):
    bsz, fields = input_batch_inputs_.shape
    num_rows, dim = weight.shape
    fpad = _round_up(fields, 8)
    idx_pad = jnp.pad(input_batch_inputs_,
                      ((0, 0), (0, fpad - fields)), mode="wrap").reshape(-1)
    emb = _make_gather(num_rows, dim, bsz, fields)(idx_pad, weight)
    mm = _make_mm(mat1.shape[0], mat1.shape[1], mat2.shape[1])(mat1, mat2)
    return emb, mm


# speedup vs baseline: 12.9247x; 1.8520x over previous
import functools

import jax
import jax.numpy as jnp
from jax import lax
from jax.experimental import pallas as pl
from jax.experimental.pallas import tpu as pltpu
from jax.experimental.pallas import tpu_sc as plsc

_NUM_CORES = 2
_NUM_SUBCORES = 16
_NW = _NUM_CORES * _NUM_SUBCORES

_NBUF = 3


@functools.lru_cache(maxsize=None)
def _make_gather(num_rows: int, dim: int, bsz: int, fields: int):
    assert bsz % _NW == 0
    bpw = bsz // _NW

    mesh = plsc.VectorSubcoreMesh(core_axis_name="c", subcore_axis_name="s")

    @functools.partial(
        pl.kernel,
        out_type=jax.ShapeDtypeStruct((fields, bsz, dim), jnp.float32),
        mesh=mesh,
        compiler_params=pltpu.CompilerParams(use_tc_tiling_on_sc=True),
        scratch_types=[
            pltpu.VMEM((fields * bpw,), jnp.int32),
            [pltpu.VMEM((bpw, dim), jnp.float32) for _ in range(_NBUF)],
            [pltpu.SemaphoreType.DMA for _ in range(_NBUF)],
            [pltpu.SemaphoreType.DMA for _ in range(_NBUF)],
            pltpu.SemaphoreType.DMA,
        ],
    )
    def gather(idx_hbm, tbl_hbm, out_hbm, idx_v, bufs, gsems, wsems, isem):
        wid = lax.axis_index("s") * _NUM_CORES + lax.axis_index("c")
        b0 = wid * bpw

        idescs = []
        for f in range(fields):
            idescs.append(pltpu.async_copy(
                idx_hbm.at[pl.ds(f * bsz + b0, bpw)],
                idx_v.at[pl.ds(f * bpw, bpw)], isem))
        for d in idescs:
            d.wait()

        def start_gather(f):
            b = f % _NBUF
            return pltpu.async_copy(
                tbl_hbm.at[idx_v.at[pl.ds(f * bpw, bpw)]], bufs[b], gsems[b])

        def write_out(f):
            b = f % _NBUF
            return pltpu.async_copy(
                bufs[b], out_hbm.at[f, pl.ds(b0, bpw)], wsems[b])

        gd = [None] * fields
        wd = [None] * fields
        for f in range(fields):
            if f >= _NBUF:
                wd[f - _NBUF].wait()
            gd[f] = start_gather(f)
            if f >= 1:
                gd[f - 1].wait()
                wd[f - 1] = write_out(f - 1)
        gd[fields - 1].wait()
        wd[fields - 1] = write_out(fields - 1)
        for f in range(max(0, fields - _NBUF + 1), fields):
            wd[f].wait()

    return gather


def _mm_body(a_ref, b_ref, o_ref):
    o_ref[...] = jnp.dot(a_ref[...], b_ref[...],
                         preferred_element_type=jnp.float32)


@functools.lru_cache(maxsize=None)
def _make_mm(m: int, k: int, n: int):
    return pl.pallas_call(
        _mm_body,
        out_shape=jax.ShapeDtypeStruct((m, n), jnp.float32),
    )


@jax.jit
def kernel(input_batch_inputs_, weight, mat1, mat2):
    bsz, fields = input_batch_inputs_.shape
    num_rows, dim = weight.shape
    idx_t = jnp.swapaxes(input_batch_inputs_, 0, 1).reshape(-1)
    emb_t = _make_gather(num_rows, dim, bsz, fields)(idx_t, weight)
    emb = jnp.transpose(emb_t, (1, 0, 2))
    mm = _make_mm(mat1.shape[0], mat1.shape[1], mat2.shape[1])(mat1, mat2)
    return emb, mm
